# Initial kernel scaffold; baseline (speedup 1.0000x reference)
import jax
import jax.numpy as jnp
from jax.experimental import pallas as pl


def kernel(x, peptide_bond_edge_index, peptide_bond_edge_attr, same_protein_edge_index, same_protein_edge_attr, interface_edge_index, interface_edge_attr, params):
    raise NotImplementedError("write your pallas kernel here")


# speedup vs baseline: 83.7957x; 83.7957x over previous
import functools

import jax
import jax.numpy as jnp
import numpy as np
from jax import lax
from jax.experimental import pallas as pl
from jax.experimental.pallas import tpu as pltpu
from jax.experimental.pallas import tpu_sc as plsc

_NC, _NS, _NL = 2, 16, 16
_NW = _NC * _NS

_SC_PARAMS = pltpu.CompilerParams(
    use_tc_tiling_on_sc=False, needs_layout_passes=False)

_EPS = 1e-16


def _pad16(n):
    return ((n + 15) // 16) * 16


def _splat(v):
    return jnp.broadcast_to(jnp.int32(v), (_NL,))


def _sc_mesh():
    return plsc.VectorSubcoreMesh(
        core_axis_name="c", subcore_axis_name="s",
        num_cores=_NC, num_subcores=_NS)


def _gat_edge_pass(xlp, xrp, srcp, dstp, attrp, wvec, *, H, C, PACC, CHUNK,
                   NCH, NACC, BR, lo, rng):
    HC = H * C
    PXL = xlp.shape[1]
    PXR = xrp.shape[1]
    NWV = wvec.shape[0]
    EW = CHUNK * NCH
    NB = NACC // _NS // BR

    @functools.partial(
        pl.kernel, mesh=_sc_mesh(),
        out_type=jax.ShapeDtypeStruct((_NC, NACC, PACC), jnp.float32),
        scratch_types=[
            pltpu.VMEM((CHUNK,), jnp.int32),
            pltpu.VMEM((CHUNK,), jnp.int32),
            pltpu.VMEM((CHUNK,), jnp.int32),
            pltpu.VMEM((CHUNK,), jnp.float32),
            pltpu.VMEM((CHUNK, PXL), jnp.float32),
            pltpu.VMEM((CHUNK, PXR), jnp.float32),
            pltpu.VMEM((CHUNK, PACC), jnp.float32),
            pltpu.VMEM((BR, PACC), jnp.float32),
            pltpu.VMEM((NWV, _NL), jnp.float32),
            pltpu.VMEM_SHARED((NACC, PACC), jnp.float32),
            pltpu.SemaphoreType.DMA,
            pltpu.SemaphoreType.DMA,
        ],
        compiler_params=_SC_PARAMS)
    def k(xl_hbm, xr_hbm, src_hbm, dst_hbm, attr_hbm, wv_hbm, out_hbm,
          src_v, dst_v, targ_v, attr_v, xlr_v, xrr_v, sc_v, bb_v, wv_v,
          acc_sh, sem1, sem2):
        cid = lax.axis_index("c")
        sid = lax.axis_index("s")
        iota = lax.iota(jnp.int32, _NL)
        zero16 = jnp.zeros((_NL,), jnp.float32)
        pltpu.sync_copy(wv_hbm, wv_v)

        def zrow(r, _):
            for j0 in range(0, PACC, 16):
                j = min(j0, PACC - 16)
                bb_v[r, pl.ds(j, 16)] = zero16
            return 0
        lax.fori_loop(0, BR, zrow, 0)

        def zrow2(r, _):
            for j0 in range(0, PACC, 16):
                j = min(j0, PACC - 16)
                sc_v[r, pl.ds(j, 16)] = zero16
            return 0
        lax.fori_loop(0, CHUNK, zrow2, 0)

        base0 = sid * (NACC // _NS)

        def zacc(i, _):
            pltpu.sync_copy(bb_v, acc_sh.at[pl.ds(base0 + i * BR, BR)])
            return 0
        lax.fori_loop(0, NB, zacc, 0)
        plsc.subcore_barrier()

        we_s = [wv_v[j, :] for j in range(HC)]
        att_s = [wv_v[HC + j, :] for j in range(HC)]

        ebase = (sid * _NC + cid) * EW

        def chunk_body(ch, _):
            eb = ebase + ch * CHUNK
            pltpu.sync_copy(src_hbm.at[pl.ds(eb, CHUNK)], src_v)
            pltpu.sync_copy(dst_hbm.at[pl.ds(eb, CHUNK)], dst_v)
            pltpu.sync_copy(attr_hbm.at[pl.ds(eb, CHUNK)], attr_v)
            cp1 = pltpu.async_copy(xl_hbm.at[src_v, :], xlr_v, sem1)
            cp2 = pltpu.async_copy(xr_hbm.at[dst_v, :], xrr_v, sem2)
            cp1.wait()
            cp2.wait()

            def tb(g, _):
                rows = iota + g * 16
                t = plsc.load_gather(dst_v, [rows])
                inr = (t >= lo) & (t < lo + rng)
                tl = jnp.where(inr, t - lo, jnp.int32(NACC - 1))
                plsc.store_scatter(targ_v, [rows], tl)
                return 0
            lax.fori_loop(0, CHUNK // 16, tb, 0)

            def gb(g, _):
                rows = iota + g * 16
                eav = plsc.load_gather(attr_v, [rows])
                for h in range(H):
                    alpha = None
                    xls = []
                    for c in range(C):
                        j = h * C + c
                        xlv = plsc.load_gather(xlr_v, [rows, _splat(j)])
                        xrv = plsc.load_gather(xrr_v, [rows, _splat(j)])
                        m = xlv + xrv + eav * we_s[j]
                        m = jnp.where(m > 0, m, 0.2 * m)
                        a = m * att_s[j]
                        alpha = a if alpha is None else alpha + a
                        xls.append(xlv)
                    ex = jnp.exp(alpha)
                    for c in range(C):
                        plsc.store_scatter(
                            sc_v, [rows, _splat(h * (C + 1) + c)], xls[c] * ex)
                    plsc.store_scatter(
                        sc_v, [rows, _splat(h * (C + 1) + C)], ex)
                return 0
            lax.fori_loop(0, CHUNK // 16, gb, 0)

            pltpu.sync_copy(sc_v, acc_sh.at[targ_v, :], add=True)
            return 0
        lax.fori_loop(0, NCH, chunk_body, 0)
        plsc.subcore_barrier()

        def dump(i, _):
            r0 = base0 + i * BR
            pltpu.sync_copy(acc_sh.at[pl.ds(r0, BR)], bb_v)
            pltpu.sync_copy(bb_v, out_hbm.at[cid, pl.ds(r0, BR)])
            return 0
        lax.fori_loop(0, NB, dump, 0)

    return k(xlp, xrp, srcp, dstp, attrp, wvec)


_BRF = 2048


def _grid_rows(n):
    return (n + _BRF - 1) // _BRF


def _mm2(y, W1, b1, W2, b2):
    n, din = y.shape
    p1 = W1.shape[1]
    p2 = W2.shape[1]

    def body(y_ref, w1_ref, b1_ref, w2_ref, b2_ref, o1_ref, o2_ref):
        yb = y_ref[...]
        o1_ref[...] = jnp.dot(yb, w1_ref[...],
                              preferred_element_type=jnp.float32) + b1_ref[...]
        o2_ref[...] = jnp.dot(yb, w2_ref[...],
                              preferred_element_type=jnp.float32) + b2_ref[...]

    return pl.pallas_call(
        body,
        grid=(_grid_rows(n),),
        in_specs=[
            pl.BlockSpec((_BRF, din), lambda r: (r, 0)),
            pl.BlockSpec((din, p1), lambda r: (0, 0)),
            pl.BlockSpec((1, p1), lambda r: (0, 0)),
            pl.BlockSpec((din, p2), lambda r: (0, 0)),
            pl.BlockSpec((1, p2), lambda r: (0, 0)),
        ],
        out_specs=[
            pl.BlockSpec((_BRF, p1), lambda r: (r, 0)),
            pl.BlockSpec((_BRF, p2), lambda r: (r, 0)),
        ],
        out_shape=[
            jax.ShapeDtypeStruct((n, p1), jnp.float32),
            jax.ShapeDtypeStruct((n, p2), jnp.float32),
        ],
    )(y, W1, b1, W2, b2)


def _mm2b(a, b, Wa1, Wb1, b1, Wa2, Wb2, b2):
    n, da = a.shape
    db = b.shape[1]
    p1 = Wa1.shape[1]
    p2 = Wa2.shape[1]

    def body(a_ref, b_ref, wa1_ref, wb1_ref, b1_ref, wa2_ref, wb2_ref, b2_ref,
             o1_ref, o2_ref):
        ab = a_ref[...]
        bb = b_ref[...]
        o1_ref[...] = (jnp.dot(ab, wa1_ref[...], preferred_element_type=jnp.float32)
                       + jnp.dot(bb, wb1_ref[...], preferred_element_type=jnp.float32)
                       + b1_ref[...])
        o2_ref[...] = (jnp.dot(ab, wa2_ref[...], preferred_element_type=jnp.float32)
                       + jnp.dot(bb, wb2_ref[...], preferred_element_type=jnp.float32)
                       + b2_ref[...])

    return pl.pallas_call(
        body,
        grid=(_grid_rows(n),),
        in_specs=[
            pl.BlockSpec((_BRF, da), lambda r: (r, 0)),
            pl.BlockSpec((_BRF, db), lambda r: (r, 0)),
            pl.BlockSpec((da, p1), lambda r: (0, 0)),
            pl.BlockSpec((db, p1), lambda r: (0, 0)),
            pl.BlockSpec((1, p1), lambda r: (0, 0)),
            pl.BlockSpec((da, p2), lambda r: (0, 0)),
            pl.BlockSpec((db, p2), lambda r: (0, 0)),
            pl.BlockSpec((1, p2), lambda r: (0, 0)),
        ],
        out_specs=[
            pl.BlockSpec((_BRF, p1), lambda r: (r, 0)),
            pl.BlockSpec((_BRF, p2), lambda r: (r, 0)),
        ],
        out_shape=[
            jax.ShapeDtypeStruct((n, p1), jnp.float32),
            jax.ShapeDtypeStruct((n, p2), jnp.float32),
        ],
    )(a, b, Wa1, Wb1, b1, Wa2, Wb2, b2)


def _finalize(A0, A1, XL, XR, epl, att, bias, S, ST, G1, G2, *, H, C, n,
              two_acc):
    HC = H * C
    PACC = A0.shape[1]
    PXL = XL.shape[1]
    PXR = XR.shape[1]
    a1_map = (lambda r: (r, 0)) if two_acc else (lambda r: (0, 0))

    def body(a0_ref, a1_ref, xl_ref, xr_ref, epl_ref, att_ref, bias_ref,
             s_ref, st_ref, g1_ref, g2_ref, o_ref):
        xl = xl_ref[:, :HC]
        xr = xr_ref[:, :HC]
        m = xl + xr + epl_ref[...]
        m = jnp.where(m > 0, m, 0.2 * m)
        alpha = jnp.dot(m * att_ref[...], s_ref[...],
                        preferred_element_type=jnp.float32)
        ex = jnp.exp(alpha)
        acc = a0_ref[...] + a1_ref[...]
        exf = jnp.dot(ex, st_ref[...], preferred_element_type=jnp.float32)
        num = jnp.dot(acc, g1_ref[...],
                      preferred_element_type=jnp.float32) + exf * xl
        den = jnp.dot(acc, g2_ref[...],
                      preferred_element_type=jnp.float32) + ex
        denf = jnp.dot(den, st_ref[...], preferred_element_type=jnp.float32)
        y = num / (denf + _EPS) + bias_ref[...]
        o_ref[...] = jnp.where(y > 0, y, 0.01 * y)

    return pl.pallas_call(
        body,
        grid=(_grid_rows(n),),
        in_specs=[
            pl.BlockSpec((_BRF, PACC), lambda r: (r, 0)),
            pl.BlockSpec((_BRF, PACC), a1_map),
            pl.BlockSpec((_BRF, PXL), lambda r: (r, 0)),
            pl.BlockSpec((_BRF, PXR), lambda r: (r, 0)),
            pl.BlockSpec((1, HC), lambda r: (0, 0)),
            pl.BlockSpec((1, HC), lambda r: (0, 0)),
            pl.BlockSpec((1, HC), lambda r: (0, 0)),
            pl.BlockSpec((HC, H), lambda r: (0, 0)),
            pl.BlockSpec((H, HC), lambda r: (0, 0)),
            pl.BlockSpec((PACC, HC), lambda r: (0, 0)),
            pl.BlockSpec((PACC, H), lambda r: (0, 0)),
        ],
        out_specs=pl.BlockSpec((_BRF, HC), lambda r: (r, 0)),
        out_shape=jax.ShapeDtypeStruct((n, HC), jnp.float32),
    )(A0, A1, XL, XR, epl, att, bias, S, ST, G1, G2)


def _iface_pass(U, V, srcp, dstp, attrp, wflat, *, CHUNK, NCH, E):
    PU = U.shape[1]
    PWC = wflat.shape[0]
    EW = CHUNK * NCH

    @functools.partial(
        pl.kernel, mesh=_sc_mesh(),
        out_type=jax.ShapeDtypeStruct((_NW, _NL), jnp.float32),
        scratch_types=[
            pltpu.VMEM((CHUNK,), jnp.int32),
            pltpu.VMEM((CHUNK,), jnp.int32),
            pltpu.VMEM((CHUNK,), jnp.float32),
            pltpu.VMEM((CHUNK, PU), jnp.float32),
            pltpu.VMEM((CHUNK, PU), jnp.float32),
            pltpu.VMEM((PWC, _NL), jnp.float32),
            pltpu.VMEM((_NL,), jnp.float32),
            pltpu.SemaphoreType.DMA,
            pltpu.SemaphoreType.DMA,
        ],
        compiler_params=_SC_PARAMS)
    def k(u_hbm, v_hbm, src_hbm, dst_hbm, attr_hbm, wc_hbm, out_hbm,
          src_v, dst_v, attr_v, ur_v, vr_v, wc_v, o_v, sem1, sem2):
        cid = lax.axis_index("c")
        sid = lax.axis_index("s")
        wid = sid * _NC + cid
        iota = lax.iota(jnp.int32, _NL)
        pltpu.sync_copy(wc_hbm, wc_v)
        ebase = wid * EW

        def w69(j):
            return wc_v[j, :]

        def w2(j, kk):
            return wc_v[32 + j * 10 + kk, :]

        def b2(kk):
            return wc_v[282 + kk, :]

        def w3(kk):
            return wc_v[292 + kk, :]

        b3 = wc_v[302, :]

        def chunk_body(ch, acc):
            eb = ebase + ch * CHUNK
            pltpu.sync_copy(src_hbm.at[pl.ds(eb, CHUNK)], src_v)
            pltpu.sync_copy(dst_hbm.at[pl.ds(eb, CHUNK)], dst_v)
            pltpu.sync_copy(attr_hbm.at[pl.ds(eb, CHUNK)], attr_v)
            cp1 = pltpu.async_copy(u_hbm.at[src_v, :], ur_v, sem1)
            cp2 = pltpu.async_copy(v_hbm.at[dst_v, :], vr_v, sem2)
            cp1.wait()
            cp2.wait()

            def gb(g, acc2):
                rows = iota + g * 16
                eav = plsc.load_gather(attr_v, [rows])
                ts = []
                for j in range(25):
                    uv = plsc.load_gather(ur_v, [rows, _splat(j)])
                    vv = plsc.load_gather(vr_v, [rows, _splat(j)])
                    t = uv + vv + eav * w69(j)
                    ts.append(jnp.where(t > 0, t, 0.01 * t))
                s = b3
                for kk in range(10):
                    z = b2(kk)
                    for j in range(25):
                        z = z + ts[j] * w2(j, kk)
                    z = jnp.where(z > 0, z, 0.01 * z)
                    s = s + z * w3(kk)
                eid = eb + g * 16 + iota
                return acc2 + jnp.where(eid < E, s, 0.0)
            return lax.fori_loop(0, CHUNK // 16, gb, acc)

        acc = lax.fori_loop(0, NCH, chunk_body, jnp.zeros((_NL,), jnp.float32))
        o_v[...] = acc
        pltpu.sync_copy(o_v, out_hbm.at[wid])

    return k(U, V, srcp, dstp, attrp, wflat)


def _final_sum(parts):
    def body(p_ref, o_ref):
        o_ref[...] = jnp.reshape(jnp.sum(p_ref[...]), (1, 1))

    return pl.pallas_call(
        body,
        out_shape=jax.ShapeDtypeStruct((1, 1), jnp.float32),
    )(parts)


def _gat_layer(y, srcp, dstp, attrp, mean_attr, p, *, H, C, PX, PACC, CHUNK,
               NCH, NACC, BR, npass, n):
    HC = H * C
    Wl = jnp.pad(p["Wl"], ((0, 0), (0, PX - HC)))
    bl = jnp.pad(p["bl"], (0, PX - HC))[None, :]
    Wr = jnp.pad(p["Wr"], ((0, 0), (0, PX - HC)))
    br = jnp.pad(p["br"], (0, PX - HC))[None, :]
    xl, xr = _mm2(y, Wl, bl, Wr, br)
    xlp = jnp.pad(xl, ((0, 1), (0, 0)))
    xrp = jnp.pad(xr, ((0, 1), (0, 0)))

    we_row = p["We"][0]
    att_row = p["att"][0].reshape(HC)
    wvec = jnp.repeat(jnp.concatenate([we_row, att_row])[:, None], _NL, axis=1)

    rng = n // npass
    accs = [_gat_edge_pass(xlp, xrp, srcp, dstp, attrp, wvec, H=H, C=C,
                           PACC=PACC, CHUNK=CHUNK, NCH=NCH, NACC=NACC, BR=BR,
                           lo=q * rng, rng=rng)
            for q in range(npass)]
    if npass == 1:
        A0, A1 = accs[0][0], accs[0][1]
    else:
        A0 = jnp.concatenate([a[0, :rng] for a in accs], axis=0)
        A1 = jnp.concatenate([a[1, :rng] for a in accs], axis=0)

    S = jnp.asarray(np.kron(np.eye(H), np.ones((C, 1))), jnp.float32)
    ST = S.T
    G1 = np.zeros((PACC, HC), np.float32)
    G2 = np.zeros((PACC, H), np.float32)
    for h in range(H):
        for c in range(C):
            G1[h * (C + 1) + c, h * C + c] = 1.0
        G2[h * (C + 1) + C, h] = 1.0
    G1 = jnp.asarray(G1)
    G2 = jnp.asarray(G2)

    epl = (mean_attr * we_row)[None, :]
    att2 = att_row[None, :]
    bias = p["bias"][None, :]
    return _finalize(A0, A1, xlp, xrp, epl, att2, bias, S, ST, G1, G2,
                     H=H, C=C, n=n, two_acc=True)


def kernel(x, peptide_bond_edge_index, peptide_bond_edge_attr,
           same_protein_edge_index, same_protein_edge_attr,
           interface_edge_index, interface_edge_attr, params):
    n = x.shape[0]

    def pad_edges(idx, attr, epad):
        e = idx.shape[1]
        src = jnp.pad(idx[0], (0, epad - e), constant_values=n)
        dst = jnp.pad(idx[1], (0, epad - e), constant_values=n)
        at = jnp.pad(attr.reshape(-1), (0, epad - e))
        return src, dst, at

    pb_src, pb_dst, pb_at = pad_edges(
        peptide_bond_edge_index, peptide_bond_edge_attr, 212992)
    sp_src, sp_dst, sp_at = pad_edges(
        same_protein_edge_index, same_protein_edge_attr, 1605632)
    if_src, if_dst, if_at = pad_edges(
        interface_edge_index, interface_edge_attr[:, None], 409600)

    pb_mean = jnp.mean(peptide_bond_edge_attr)
    sp_mean = jnp.mean(same_protein_edge_attr)

    y = _gat_layer(x, pb_src, pb_dst, pb_at, pb_mean, params["pc1"],
                   H=2, C=5, PX=16, PACC=16, CHUNK=256, NCH=26,
                   NACC=102400, BR=256, npass=1, n=n)
    y = _gat_layer(y, pb_src, pb_dst, pb_at, pb_mean, params["pc2"],
                   H=3, C=5, PX=16, PACC=24, CHUNK=256, NCH=26,
                   NACC=53248, BR=256, npass=2, n=n)
    px = _gat_layer(y, pb_src, pb_dst, pb_at, pb_mean, params["pc3"],
                    H=3, C=10, PX=32, PACC=40, CHUNK=256, NCH=26,
                    NACC=28672, BR=256, npass=4, n=n)
    y = _gat_layer(px, sp_src, sp_dst, sp_at, sp_mean, params["prc1"],
                   H=2, C=2, PX=16, PACC=16, CHUNK=256, NCH=196,
                   NACC=102400, BR=256, npass=1, n=n)
    prx = _gat_layer(y, sp_src, sp_dst, sp_at, sp_mean, params["prc2"],
                     H=2, C=2, PX=16, PACC=16, CHUNK=256, NCH=196,
                     NACC=102400, BR=256, npass=1, n=n)

    e1W = params["e1W"]
    Wa1 = jnp.pad(e1W[0:30], ((0, 0), (0, 7)))
    Wb1 = jnp.pad(e1W[30:34], ((0, 0), (0, 7)))
    b1 = jnp.pad(params["e1b"], (0, 7))[None, :]
    Wa2 = jnp.pad(e1W[34:64], ((0, 0), (0, 7)))
    Wb2 = jnp.pad(e1W[64:68], ((0, 0), (0, 7)))
    bz = jnp.zeros((1, 32), jnp.float32)
    U, V = _mm2b(px, prx, Wa1, Wb1, b1, Wa2, Wb2, bz)
    Up = jnp.pad(U, ((0, 1), (0, 0)))
    Vp = jnp.pad(V, ((0, 1), (0, 0)))

    wflat = jnp.zeros((304,), jnp.float32)
    wflat = wflat.at[0:25].set(e1W[68])
    wflat = wflat.at[32:282].set(params["e2W"].reshape(-1))
    wflat = wflat.at[282:292].set(params["e2b"])
    wflat = wflat.at[292:302].set(params["e3W"][:, 0])
    wflat = wflat.at[302].set(params["e3b"][0])
    wflat = jnp.repeat(wflat[:, None], _NL, axis=1)

    parts = _iface_pass(Up, Vp, if_src, if_dst, if_at, wflat,
                        CHUNK=512, NCH=25, E=interface_edge_index.shape[1])
    return _final_sum(parts)


# speedup vs baseline: 96.0105x; 1.1458x over previous
import functools

import jax
import jax.numpy as jnp
import numpy as np
from jax import lax
from jax.experimental import pallas as pl
from jax.experimental.pallas import tpu as pltpu
from jax.experimental.pallas import tpu_sc as plsc

_NC, _NS, _NL = 2, 16, 16
_NW = _NC * _NS

_SC_PARAMS = pltpu.CompilerParams(
    use_tc_tiling_on_sc=False, needs_layout_passes=False)

_EPS = 1e-16


def _pad16(n):
    return ((n + 15) // 16) * 16


def _splat(v):
    return jnp.broadcast_to(jnp.int32(v), (_NL,))


def _sc_mesh():
    return plsc.VectorSubcoreMesh(
        core_axis_name="c", subcore_axis_name="s",
        num_cores=_NC, num_subcores=_NS)


def _gat_edge_pass(xlp, xrp, srcp, dstp, attrp, wvec, *, H, C, PACC, CHUNK,
                   NCH, NACC, BR, lo, rng):
    HC = H * C
    PXL = xlp.shape[1]
    PXR = xrp.shape[1]
    NWV = wvec.shape[0]
    EW = CHUNK * NCH
    NB = NACC // _NS // BR

    @functools.partial(
        pl.kernel, mesh=_sc_mesh(),
        out_type=jax.ShapeDtypeStruct((_NC, NACC, PACC), jnp.float32),
        scratch_types=[
            [pltpu.VMEM((CHUNK,), jnp.int32)] * 2,
            [pltpu.VMEM((CHUNK,), jnp.int32)] * 2,
            pltpu.VMEM((CHUNK,), jnp.int32),
            [pltpu.VMEM((CHUNK,), jnp.float32)] * 2,
            [pltpu.VMEM((CHUNK, PXL), jnp.float32)] * 2,
            [pltpu.VMEM((CHUNK, PXR), jnp.float32)] * 2,
            pltpu.VMEM((CHUNK, PACC), jnp.float32),
            pltpu.VMEM((BR, PACC), jnp.float32),
            pltpu.VMEM((NWV, _NL), jnp.float32),
            pltpu.VMEM_SHARED((NACC, PACC), jnp.float32),
            [pltpu.SemaphoreType.DMA] * 2,
            [pltpu.SemaphoreType.DMA] * 2,
        ],
        compiler_params=_SC_PARAMS)
    def k(xl_hbm, xr_hbm, src_hbm, dst_hbm, attr_hbm, wv_hbm, out_hbm,
          src_v, dst_v, targ_v, attr_v, xlr_v, xrr_v, sc_v, bb_v, wv_v,
          acc_sh, sem1, sem2):
        cid = lax.axis_index("c")
        sid = lax.axis_index("s")
        iota = lax.iota(jnp.int32, _NL)
        zero16 = jnp.zeros((_NL,), jnp.float32)
        pltpu.sync_copy(wv_hbm, wv_v)

        def zrow(r, _):
            for j0 in range(0, PACC, 16):
                j = min(j0, PACC - 16)
                bb_v[r, pl.ds(j, 16)] = zero16
            return 0
        lax.fori_loop(0, BR, zrow, 0)

        def zrow2(r, _):
            for j0 in range(0, PACC, 16):
                j = min(j0, PACC - 16)
                sc_v[r, pl.ds(j, 16)] = zero16
            return 0
        lax.fori_loop(0, CHUNK, zrow2, 0)

        base0 = sid * (NACC // _NS)

        def zacc(i, _):
            pltpu.sync_copy(bb_v, acc_sh.at[pl.ds(base0 + i * BR, BR)])
            return 0
        lax.fori_loop(0, NB, zacc, 0)
        plsc.subcore_barrier()

        we_s = [wv_v[j, :] for j in range(HC)]
        att_s = [wv_v[HC + j, :] for j in range(HC)]

        ebase = (sid * _NC + cid) * EW

        def fire(ch, b):
            eb = ebase + ch * CHUNK
            pltpu.sync_copy(src_hbm.at[pl.ds(eb, CHUNK)], src_v[b])
            pltpu.sync_copy(dst_hbm.at[pl.ds(eb, CHUNK)], dst_v[b])
            pltpu.sync_copy(attr_hbm.at[pl.ds(eb, CHUNK)], attr_v[b])
            pltpu.async_copy(xl_hbm.at[src_v[b], :], xlr_v[b], sem1[b])
            pltpu.async_copy(xr_hbm.at[dst_v[b], :], xrr_v[b], sem2[b])

        def consume(b):
            pltpu.make_async_copy(xl_hbm.at[src_v[b], :], xlr_v[b],
                                  sem1[b]).wait()
            pltpu.make_async_copy(xr_hbm.at[dst_v[b], :], xrr_v[b],
                                  sem2[b]).wait()

            def tb(g, _):
                rows = iota + g * 16
                t = plsc.load_gather(dst_v[b], [rows])
                inr = (t >= lo) & (t < lo + rng)
                tl = jnp.where(inr, t - lo, jnp.int32(NACC - 1))
                plsc.store_scatter(targ_v, [rows], tl)
                return 0
            lax.fori_loop(0, CHUNK // 16, tb, 0)

            def gb(g, _):
                rows = iota + g * 16
                eav = plsc.load_gather(attr_v[b], [rows])
                for h in range(H):
                    alpha = None
                    xls = []
                    for c in range(C):
                        j = h * C + c
                        xlv = plsc.load_gather(xlr_v[b], [rows, _splat(j)])
                        xrv = plsc.load_gather(xrr_v[b], [rows, _splat(j)])
                        m = xlv + xrv + eav * we_s[j]
                        m = jnp.where(m > 0, m, 0.2 * m)
                        a = m * att_s[j]
                        alpha = a if alpha is None else alpha + a
                        xls.append(xlv)
                    ex = jnp.exp(alpha)
                    for c in range(C):
                        plsc.store_scatter(
                            sc_v, [rows, _splat(h * (C + 1) + c)], xls[c] * ex)
                    plsc.store_scatter(
                        sc_v, [rows, _splat(h * (C + 1) + C)], ex)
                return 0
            lax.fori_loop(0, CHUNK // 16, gb, 0)

            pltpu.sync_copy(sc_v, acc_sh.at[targ_v, :], add=True)

        fire(0, 0)
        fire(1, 1)

        def chunk_pair(i, _):
            a = 2 * i
            consume(0)
            fire(a + 2, 0)
            consume(1)
            fire(a + 3, 1)
            return 0
        lax.fori_loop(0, NCH // 2 - 1, chunk_pair, 0)
        consume(0)
        consume(1)
        plsc.subcore_barrier()

        def dump(i, _):
            r0 = base0 + i * BR
            pltpu.sync_copy(acc_sh.at[pl.ds(r0, BR)], bb_v)
            pltpu.sync_copy(bb_v, out_hbm.at[cid, pl.ds(r0, BR)])
            return 0
        lax.fori_loop(0, NB, dump, 0)

    return k(xlp, xrp, srcp, dstp, attrp, wvec)


_BRF = 2048


def _grid_rows(n):
    return (n + _BRF - 1) // _BRF


def _mm2(y, W1, b1, W2, b2):
    n, din = y.shape
    p1 = W1.shape[1]
    p2 = W2.shape[1]

    def body(y_ref, w1_ref, b1_ref, w2_ref, b2_ref, o1_ref, o2_ref):
        yb = y_ref[...]
        o1_ref[...] = jnp.dot(yb, w1_ref[...],
                              preferred_element_type=jnp.float32) + b1_ref[...]
        o2_ref[...] = jnp.dot(yb, w2_ref[...],
                              preferred_element_type=jnp.float32) + b2_ref[...]

    return pl.pallas_call(
        body,
        grid=(_grid_rows(n),),
        in_specs=[
            pl.BlockSpec((_BRF, din), lambda r: (r, 0)),
            pl.BlockSpec((din, p1), lambda r: (0, 0)),
            pl.BlockSpec((1, p1), lambda r: (0, 0)),
            pl.BlockSpec((din, p2), lambda r: (0, 0)),
            pl.BlockSpec((1, p2), lambda r: (0, 0)),
        ],
        out_specs=[
            pl.BlockSpec((_BRF, p1), lambda r: (r, 0)),
            pl.BlockSpec((_BRF, p2), lambda r: (r, 0)),
        ],
        out_shape=[
            jax.ShapeDtypeStruct((n, p1), jnp.float32),
            jax.ShapeDtypeStruct((n, p2), jnp.float32),
        ],
    )(y, W1, b1, W2, b2)


def _mm2b(a, b, Wa1, Wb1, b1, Wa2, Wb2, b2):
    n, da = a.shape
    db = b.shape[1]
    p1 = Wa1.shape[1]
    p2 = Wa2.shape[1]

    def body(a_ref, b_ref, wa1_ref, wb1_ref, b1_ref, wa2_ref, wb2_ref, b2_ref,
             o1_ref, o2_ref):
        ab = a_ref[...]
        bb = b_ref[...]
        o1_ref[...] = (jnp.dot(ab, wa1_ref[...], preferred_element_type=jnp.float32)
                       + jnp.dot(bb, wb1_ref[...], preferred_element_type=jnp.float32)
                       + b1_ref[...])
        o2_ref[...] = (jnp.dot(ab, wa2_ref[...], preferred_element_type=jnp.float32)
                       + jnp.dot(bb, wb2_ref[...], preferred_element_type=jnp.float32)
                       + b2_ref[...])

    return pl.pallas_call(
        body,
        grid=(_grid_rows(n),),
        in_specs=[
            pl.BlockSpec((_BRF, da), lambda r: (r, 0)),
            pl.BlockSpec((_BRF, db), lambda r: (r, 0)),
            pl.BlockSpec((da, p1), lambda r: (0, 0)),
            pl.BlockSpec((db, p1), lambda r: (0, 0)),
            pl.BlockSpec((1, p1), lambda r: (0, 0)),
            pl.BlockSpec((da, p2), lambda r: (0, 0)),
            pl.BlockSpec((db, p2), lambda r: (0, 0)),
            pl.BlockSpec((1, p2), lambda r: (0, 0)),
        ],
        out_specs=[
            pl.BlockSpec((_BRF, p1), lambda r: (r, 0)),
            pl.BlockSpec((_BRF, p2), lambda r: (r, 0)),
        ],
        out_shape=[
            jax.ShapeDtypeStruct((n, p1), jnp.float32),
            jax.ShapeDtypeStruct((n, p2), jnp.float32),
        ],
    )(a, b, Wa1, Wb1, b1, Wa2, Wb2, b2)


def _finalize(A0, A1, XL, XR, epl, att, bias, S, ST, G1, G2, *, H, C, n,
              two_acc):
    HC = H * C
    PACC = A0.shape[1]
    PXL = XL.shape[1]
    PXR = XR.shape[1]
    a1_map = (lambda r: (r, 0)) if two_acc else (lambda r: (0, 0))

    def body(a0_ref, a1_ref, xl_ref, xr_ref, epl_ref, att_ref, bias_ref,
             s_ref, st_ref, g1_ref, g2_ref, o_ref):
        xl = xl_ref[:, :HC]
        xr = xr_ref[:, :HC]
        m = xl + xr + epl_ref[...]
        m = jnp.where(m > 0, m, 0.2 * m)
        alpha = jnp.dot(m * att_ref[...], s_ref[...],
                        preferred_element_type=jnp.float32)
        ex = jnp.exp(alpha)
        acc = a0_ref[...] + a1_ref[...]
        exf = jnp.dot(ex, st_ref[...], preferred_element_type=jnp.float32)
        num = jnp.dot(acc, g1_ref[...],
                      preferred_element_type=jnp.float32) + exf * xl
        den = jnp.dot(acc, g2_ref[...],
                      preferred_element_type=jnp.float32) + ex
        denf = jnp.dot(den, st_ref[...], preferred_element_type=jnp.float32)
        y = num / (denf + _EPS) + bias_ref[...]
        o_ref[...] = jnp.where(y > 0, y, 0.01 * y)

    return pl.pallas_call(
        body,
        grid=(_grid_rows(n),),
        in_specs=[
            pl.BlockSpec((_BRF, PACC), lambda r: (r, 0)),
            pl.BlockSpec((_BRF, PACC), a1_map),
            pl.BlockSpec((_BRF, PXL), lambda r: (r, 0)),
            pl.BlockSpec((_BRF, PXR), lambda r: (r, 0)),
            pl.BlockSpec((1, HC), lambda r: (0, 0)),
            pl.BlockSpec((1, HC), lambda r: (0, 0)),
            pl.BlockSpec((1, HC), lambda r: (0, 0)),
            pl.BlockSpec((HC, H), lambda r: (0, 0)),
            pl.BlockSpec((H, HC), lambda r: (0, 0)),
            pl.BlockSpec((PACC, HC), lambda r: (0, 0)),
            pl.BlockSpec((PACC, H), lambda r: (0, 0)),
        ],
        out_specs=pl.BlockSpec((_BRF, HC), lambda r: (r, 0)),
        out_shape=jax.ShapeDtypeStruct((n, HC), jnp.float32),
    )(A0, A1, XL, XR, epl, att, bias, S, ST, G1, G2)


def _iface_pass(U, V, srcp, dstp, attrp, wflat, *, CHUNK, NCH, E):
    PU = U.shape[1]
    PWC = wflat.shape[0]
    EW = CHUNK * NCH

    @functools.partial(
        pl.kernel, mesh=_sc_mesh(),
        out_type=jax.ShapeDtypeStruct((_NW, _NL), jnp.float32),
        scratch_types=[
            pltpu.VMEM((CHUNK,), jnp.int32),
            pltpu.VMEM((CHUNK,), jnp.int32),
            pltpu.VMEM((CHUNK,), jnp.float32),
            pltpu.VMEM((CHUNK, PU), jnp.float32),
            pltpu.VMEM((CHUNK, PU), jnp.float32),
            pltpu.VMEM((PWC, _NL), jnp.float32),
            pltpu.VMEM((_NL,), jnp.float32),
            pltpu.SemaphoreType.DMA,
            pltpu.SemaphoreType.DMA,
        ],
        compiler_params=_SC_PARAMS)
    def k(u_hbm, v_hbm, src_hbm, dst_hbm, attr_hbm, wc_hbm, out_hbm,
          src_v, dst_v, attr_v, ur_v, vr_v, wc_v, o_v, sem1, sem2):
        cid = lax.axis_index("c")
        sid = lax.axis_index("s")
        wid = sid * _NC + cid
        iota = lax.iota(jnp.int32, _NL)
        pltpu.sync_copy(wc_hbm, wc_v)
        ebase = wid * EW

        def w69(j):
            return wc_v[j, :]

        def w2(j, kk):
            return wc_v[32 + j * 10 + kk, :]

        def b2(kk):
            return wc_v[282 + kk, :]

        def w3(kk):
            return wc_v[292 + kk, :]

        b3 = wc_v[302, :]

        def chunk_body(ch, acc):
            eb = ebase + ch * CHUNK
            pltpu.sync_copy(src_hbm.at[pl.ds(eb, CHUNK)], src_v)
            pltpu.sync_copy(dst_hbm.at[pl.ds(eb, CHUNK)], dst_v)
            pltpu.sync_copy(attr_hbm.at[pl.ds(eb, CHUNK)], attr_v)
            cp1 = pltpu.async_copy(u_hbm.at[src_v, :], ur_v, sem1)
            cp2 = pltpu.async_copy(v_hbm.at[dst_v, :], vr_v, sem2)
            cp1.wait()
            cp2.wait()

            def gb(g, acc2):
                rows = iota + g * 16
                eav = plsc.load_gather(attr_v, [rows])
                ts = []
                for j in range(25):
                    uv = plsc.load_gather(ur_v, [rows, _splat(j)])
                    vv = plsc.load_gather(vr_v, [rows, _splat(j)])
                    t = uv + vv + eav * w69(j)
                    ts.append(jnp.where(t > 0, t, 0.01 * t))
                s = b3
                for kk in range(10):
                    z = b2(kk)
                    for j in range(25):
                        z = z + ts[j] * w2(j, kk)
                    z = jnp.where(z > 0, z, 0.01 * z)
                    s = s + z * w3(kk)
                eid = eb + g * 16 + iota
                return acc2 + jnp.where(eid < E, s, 0.0)
            return lax.fori_loop(0, CHUNK // 16, gb, acc)

        acc = lax.fori_loop(0, NCH, chunk_body, jnp.zeros((_NL,), jnp.float32))
        o_v[...] = acc
        pltpu.sync_copy(o_v, out_hbm.at[wid])

    return k(U, V, srcp, dstp, attrp, wflat)


def _final_sum(parts):
    def body(p_ref, o_ref):
        o_ref[...] = jnp.reshape(jnp.sum(p_ref[...]), (1, 1))

    return pl.pallas_call(
        body,
        out_shape=jax.ShapeDtypeStruct((1, 1), jnp.float32),
    )(parts)


def _gat_layer(y, srcp, dstp, attrp, mean_attr, p, *, H, C, PX, PACC, CHUNK,
               NCH, NACC, BR, npass, n):
    HC = H * C
    Wl = jnp.pad(p["Wl"], ((0, 0), (0, PX - HC)))
    bl = jnp.pad(p["bl"], (0, PX - HC))[None, :]
    Wr = jnp.pad(p["Wr"], ((0, 0), (0, PX - HC)))
    br = jnp.pad(p["br"], (0, PX - HC))[None, :]
    xl, xr = _mm2(y, Wl, bl, Wr, br)
    xlp = jnp.pad(xl, ((0, 1), (0, 0)))
    xrp = jnp.pad(xr, ((0, 1), (0, 0)))

    we_row = p["We"][0]
    att_row = p["att"][0].reshape(HC)
    wvec = jnp.repeat(jnp.concatenate([we_row, att_row])[:, None], _NL, axis=1)

    rng = n // npass
    accs = [_gat_edge_pass(xlp, xrp, srcp, dstp, attrp, wvec, H=H, C=C,
                           PACC=PACC, CHUNK=CHUNK, NCH=NCH, NACC=NACC, BR=BR,
                           lo=q * rng, rng=rng)
            for q in range(npass)]
    if npass == 1:
        A0, A1 = accs[0][0], accs[0][1]
    else:
        A0 = jnp.concatenate([a[0, :rng] for a in accs], axis=0)
        A1 = jnp.concatenate([a[1, :rng] for a in accs], axis=0)

    S = jnp.asarray(np.kron(np.eye(H), np.ones((C, 1))), jnp.float32)
    ST = S.T
    G1 = np.zeros((PACC, HC), np.float32)
    G2 = np.zeros((PACC, H), np.float32)
    for h in range(H):
        for c in range(C):
            G1[h * (C + 1) + c, h * C + c] = 1.0
        G2[h * (C + 1) + C, h] = 1.0
    G1 = jnp.asarray(G1)
    G2 = jnp.asarray(G2)

    epl = (mean_attr * we_row)[None, :]
    att2 = att_row[None, :]
    bias = p["bias"][None, :]
    return _finalize(A0, A1, xlp, xrp, epl, att2, bias, S, ST, G1, G2,
                     H=H, C=C, n=n, two_acc=True)


def kernel(x, peptide_bond_edge_index, peptide_bond_edge_attr,
           same_protein_edge_index, same_protein_edge_attr,
           interface_edge_index, interface_edge_attr, params):
    n = x.shape[0]

    def pad_edges(idx, attr, epad):
        e = idx.shape[1]
        src = jnp.pad(idx[0], (0, epad - e), constant_values=n)
        dst = jnp.pad(idx[1], (0, epad - e), constant_values=n)
        at = jnp.pad(attr.reshape(-1), (0, epad - e))
        return src, dst, at

    pb_src, pb_dst, pb_at = pad_edges(
        peptide_bond_edge_index, peptide_bond_edge_attr, 212992)
    sp_src, sp_dst, sp_at = pad_edges(
        same_protein_edge_index, same_protein_edge_attr, 1605632)
    if_src, if_dst, if_at = pad_edges(
        interface_edge_index, interface_edge_attr[:, None], 409600)

    pb_mean = jnp.mean(peptide_bond_edge_attr)
    sp_mean = jnp.mean(same_protein_edge_attr)

    y = _gat_layer(x, pb_src, pb_dst, pb_at, pb_mean, params["pc1"],
                   H=2, C=5, PX=16, PACC=16, CHUNK=256, NCH=26,
                   NACC=102400, BR=256, npass=1, n=n)
    y = _gat_layer(y, pb_src, pb_dst, pb_at, pb_mean, params["pc2"],
                   H=3, C=5, PX=16, PACC=24, CHUNK=256, NCH=26,
                   NACC=53248, BR=256, npass=2, n=n)
    px = _gat_layer(y, pb_src, pb_dst, pb_at, pb_mean, params["pc3"],
                    H=3, C=10, PX=32, PACC=40, CHUNK=256, NCH=26,
                    NACC=28672, BR=256, npass=4, n=n)
    y = _gat_layer(px, sp_src, sp_dst, sp_at, sp_mean, params["prc1"],
                   H=2, C=2, PX=16, PACC=16, CHUNK=256, NCH=196,
                   NACC=102400, BR=256, npass=1, n=n)
    prx = _gat_layer(y, sp_src, sp_dst, sp_at, sp_mean, params["prc2"],
                     H=2, C=2, PX=16, PACC=16, CHUNK=256, NCH=196,
                     NACC=102400, BR=256, npass=1, n=n)

    e1W = params["e1W"]
    Wa1 = jnp.pad(e1W[0:30], ((0, 0), (0, 7)))
    Wb1 = jnp.pad(e1W[30:34], ((0, 0), (0, 7)))
    b1 = jnp.pad(params["e1b"], (0, 7))[None, :]
    Wa2 = jnp.pad(e1W[34:64], ((0, 0), (0, 7)))
    Wb2 = jnp.pad(e1W[64:68], ((0, 0), (0, 7)))
    bz = jnp.zeros((1, 32), jnp.float32)
    U, V = _mm2b(px, prx, Wa1, Wb1, b1, Wa2, Wb2, bz)
    Up = jnp.pad(U, ((0, 1), (0, 0)))
    Vp = jnp.pad(V, ((0, 1), (0, 0)))

    wflat = jnp.zeros((304,), jnp.float32)
    wflat = wflat.at[0:25].set(e1W[68])
    wflat = wflat.at[32:282].set(params["e2W"].reshape(-1))
    wflat = wflat.at[282:292].set(params["e2b"])
    wflat = wflat.at[292:302].set(params["e3W"][:, 0])
    wflat = wflat.at[302].set(params["e3b"][0])
    wflat = jnp.repeat(wflat[:, None], _NL, axis=1)

    parts = _iface_pass(Up, Vp, if_src, if_dst, if_at, wflat,
                        CHUNK=512, NCH=25, E=interface_edge_index.shape[1])
    return _final_sum(parts)


# speedup vs baseline: 99.6647x; 1.0381x over previous
import functools

import jax
import jax.numpy as jnp
import numpy as np
from jax import lax
from jax.experimental import pallas as pl
from jax.experimental.pallas import tpu as pltpu
from jax.experimental.pallas import tpu_sc as plsc

_NC, _NS, _NL = 2, 16, 16
_NW = _NC * _NS

_SC_PARAMS = pltpu.CompilerParams(
    use_tc_tiling_on_sc=False, needs_layout_passes=False)

_EPS = 1e-16


def _pad16(n):
    return ((n + 15) // 16) * 16


def _splat(v):
    return jnp.broadcast_to(jnp.int32(v), (_NL,))


def _sc_mesh():
    return plsc.VectorSubcoreMesh(
        core_axis_name="c", subcore_axis_name="s",
        num_cores=_NC, num_subcores=_NS)


def _gat_edge_pass(xlp, xrp, srcp, dstp, attrp, wvec, *, H, C, PACC, CHUNK,
                   NCH, NACC, BR, lo, rng):
    HC = H * C
    PXL = xlp.shape[1]
    PXR = xrp.shape[1]
    NWV = wvec.shape[0]
    EW = CHUNK * NCH
    NB = NACC // _NS // BR

    @functools.partial(
        pl.kernel, mesh=_sc_mesh(),
        out_type=jax.ShapeDtypeStruct((_NC, NACC, PACC), jnp.float32),
        scratch_types=[
            [pltpu.VMEM((CHUNK,), jnp.int32)] * 2,
            [pltpu.VMEM((CHUNK,), jnp.int32)] * 2,
            [pltpu.VMEM((CHUNK,), jnp.int32)] * 2,
            [pltpu.VMEM((CHUNK,), jnp.float32)] * 2,
            [pltpu.VMEM((CHUNK, PXL), jnp.float32)] * 2,
            [pltpu.VMEM((CHUNK, PXR), jnp.float32)] * 2,
            [pltpu.VMEM((CHUNK, PACC), jnp.float32)] * 2,
            pltpu.VMEM((BR, PACC), jnp.float32),
            pltpu.VMEM((NWV, _NL), jnp.float32),
            pltpu.VMEM_SHARED((NACC, PACC), jnp.float32),
            [pltpu.SemaphoreType.DMA] * 2,
            [pltpu.SemaphoreType.DMA] * 2,
            [pltpu.SemaphoreType.DMA] * 2,
        ],
        compiler_params=_SC_PARAMS)
    def k(xl_hbm, xr_hbm, src_hbm, dst_hbm, attr_hbm, wv_hbm, out_hbm,
          src_v, dst_v, targ_v, attr_v, xlr_v, xrr_v, sc_v, bb_v, wv_v,
          acc_sh, sem1, sem2, sem3):
        cid = lax.axis_index("c")
        sid = lax.axis_index("s")
        iota = lax.iota(jnp.int32, _NL)
        zero16 = jnp.zeros((_NL,), jnp.float32)
        pltpu.sync_copy(wv_hbm, wv_v)

        def zrow(r, _):
            for j0 in range(0, PACC, 16):
                j = min(j0, PACC - 16)
                bb_v[r, pl.ds(j, 16)] = zero16
            return 0
        lax.fori_loop(0, BR, zrow, 0)

        def zrow2(r, _):
            for b in range(2):
                for j0 in range(0, PACC, 16):
                    j = min(j0, PACC - 16)
                    sc_v[b][r, pl.ds(j, 16)] = zero16
            return 0
        lax.fori_loop(0, CHUNK, zrow2, 0)

        base0 = sid * (NACC // _NS)

        def zacc(i, _):
            pltpu.sync_copy(bb_v, acc_sh.at[pl.ds(base0 + i * BR, BR)])
            return 0
        lax.fori_loop(0, NB, zacc, 0)
        plsc.subcore_barrier()

        def we_s(j):
            return wv_v[j, :]

        def att_s(j):
            return wv_v[HC + j, :]

        ebase = (sid * _NC + cid) * EW

        def fire(ch, b):
            eb = ebase + ch * CHUNK
            pltpu.sync_copy(src_hbm.at[pl.ds(eb, CHUNK)], src_v[b])
            pltpu.sync_copy(dst_hbm.at[pl.ds(eb, CHUNK)], dst_v[b])
            pltpu.sync_copy(attr_hbm.at[pl.ds(eb, CHUNK)], attr_v[b])
            pltpu.async_copy(xl_hbm.at[src_v[b], :], xlr_v[b], sem1[b])
            pltpu.async_copy(xr_hbm.at[dst_v[b], :], xrr_v[b], sem2[b])

        def waitsc(b):
            pltpu.make_async_copy(sc_v[b], acc_sh.at[targ_v[b], :],
                                  sem3[b]).wait()

        def consume(b):
            pltpu.make_async_copy(xl_hbm.at[src_v[b], :], xlr_v[b],
                                  sem1[b]).wait()
            pltpu.make_async_copy(xr_hbm.at[dst_v[b], :], xrr_v[b],
                                  sem2[b]).wait()

            def tb(g, _):
                rows = iota + g * 16
                t = plsc.load_gather(dst_v[b], [rows])
                inr = (t >= lo) & (t < lo + rng)
                tl = jnp.where(inr, t - lo, jnp.int32(NACC - 1))
                plsc.store_scatter(targ_v[b], [rows], tl)
                return 0
            lax.fori_loop(0, CHUNK // 16, tb, 0)

            def gb(g, _):
                rows = iota + g * 16
                eav = plsc.load_gather(attr_v[b], [rows])
                for h in range(H):
                    alpha = None
                    xls = []
                    for c in range(C):
                        j = h * C + c
                        xlv = plsc.load_gather(xlr_v[b], [rows, _splat(j)])
                        xrv = plsc.load_gather(xrr_v[b], [rows, _splat(j)])
                        m = xlv + xrv + eav * we_s(j)
                        m = jnp.where(m > 0, m, 0.2 * m)
                        a = m * att_s(j)
                        alpha = a if alpha is None else alpha + a
                        xls.append(xlv)
                    ex = jnp.exp(alpha)
                    for c in range(C):
                        plsc.store_scatter(
                            sc_v[b], [rows, _splat(h * (C + 1) + c)],
                            xls[c] * ex)
                    plsc.store_scatter(
                        sc_v[b], [rows, _splat(h * (C + 1) + C)], ex)
                return 0
            lax.fori_loop(0, CHUNK // 16, gb, 0)

            pltpu.async_copy(sc_v[b], acc_sh.at[targ_v[b], :], sem3[b],
                             add=True)

        fire(0, 0)
        fire(1, 1)
        consume(0)
        fire(2, 0)
        consume(1)
        fire(3, 1)

        def chunk_pair(i, _):
            a = 2 * i
            waitsc(0)
            consume(0)
            fire(a + 2, 0)
            waitsc(1)
            consume(1)
            fire(a + 3, 1)
            return 0
        lax.fori_loop(1, NCH // 2 - 1, chunk_pair, 0)
        waitsc(0)
        consume(0)
        waitsc(1)
        consume(1)
        waitsc(0)
        waitsc(1)
        plsc.subcore_barrier()

        def dump(i, _):
            r0 = base0 + i * BR
            pltpu.sync_copy(acc_sh.at[pl.ds(r0, BR)], bb_v)
            pltpu.sync_copy(bb_v, out_hbm.at[cid, pl.ds(r0, BR)])
            return 0
        lax.fori_loop(0, NB, dump, 0)

    return k(xlp, xrp, srcp, dstp, attrp, wvec)


_BRF = 2048


def _grid_rows(n):
    return (n + _BRF - 1) // _BRF


def _mm2(y, W1, b1, W2, b2):
    n, din = y.shape
    p1 = W1.shape[1]
    p2 = W2.shape[1]

    def body(y_ref, w1_ref, b1_ref, w2_ref, b2_ref, o1_ref, o2_ref):
        yb = y_ref[...]
        o1_ref[...] = jnp.dot(yb, w1_ref[...],
                              preferred_element_type=jnp.float32) + b1_ref[...]
        o2_ref[...] = jnp.dot(yb, w2_ref[...],
                              preferred_element_type=jnp.float32) + b2_ref[...]

    return pl.pallas_call(
        body,
        grid=(_grid_rows(n),),
        in_specs=[
            pl.BlockSpec((_BRF, din), lambda r: (r, 0)),
            pl.BlockSpec((din, p1), lambda r: (0, 0)),
            pl.BlockSpec((1, p1), lambda r: (0, 0)),
            pl.BlockSpec((din, p2), lambda r: (0, 0)),
            pl.BlockSpec((1, p2), lambda r: (0, 0)),
        ],
        out_specs=[
            pl.BlockSpec((_BRF, p1), lambda r: (r, 0)),
            pl.BlockSpec((_BRF, p2), lambda r: (r, 0)),
        ],
        out_shape=[
            jax.ShapeDtypeStruct((n, p1), jnp.float32),
            jax.ShapeDtypeStruct((n, p2), jnp.float32),
        ],
    )(y, W1, b1, W2, b2)


def _mm2b(a, b, Wa1, Wb1, b1, Wa2, Wb2, b2):
    n, da = a.shape
    db = b.shape[1]
    p1 = Wa1.shape[1]
    p2 = Wa2.shape[1]

    def body(a_ref, b_ref, wa1_ref, wb1_ref, b1_ref, wa2_ref, wb2_ref, b2_ref,
             o1_ref, o2_ref):
        ab = a_ref[...]
        bb = b_ref[...]
        o1_ref[...] = (jnp.dot(ab, wa1_ref[...], preferred_element_type=jnp.float32)
                       + jnp.dot(bb, wb1_ref[...], preferred_element_type=jnp.float32)
                       + b1_ref[...])
        o2_ref[...] = (jnp.dot(ab, wa2_ref[...], preferred_element_type=jnp.float32)
                       + jnp.dot(bb, wb2_ref[...], preferred_element_type=jnp.float32)
                       + b2_ref[...])

    return pl.pallas_call(
        body,
        grid=(_grid_rows(n),),
        in_specs=[
            pl.BlockSpec((_BRF, da), lambda r: (r, 0)),
            pl.BlockSpec((_BRF, db), lambda r: (r, 0)),
            pl.BlockSpec((da, p1), lambda r: (0, 0)),
            pl.BlockSpec((db, p1), lambda r: (0, 0)),
            pl.BlockSpec((1, p1), lambda r: (0, 0)),
            pl.BlockSpec((da, p2), lambda r: (0, 0)),
            pl.BlockSpec((db, p2), lambda r: (0, 0)),
            pl.BlockSpec((1, p2), lambda r: (0, 0)),
        ],
        out_specs=[
            pl.BlockSpec((_BRF, p1), lambda r: (r, 0)),
            pl.BlockSpec((_BRF, p2), lambda r: (r, 0)),
        ],
        out_shape=[
            jax.ShapeDtypeStruct((n, p1), jnp.float32),
            jax.ShapeDtypeStruct((n, p2), jnp.float32),
        ],
    )(a, b, Wa1, Wb1, b1, Wa2, Wb2, b2)


def _finalize(A0, A1, XL, XR, epl, att, bias, S, ST, G1, G2, *, H, C, n,
              two_acc):
    HC = H * C
    PACC = A0.shape[1]
    PXL = XL.shape[1]
    PXR = XR.shape[1]
    a1_map = (lambda r: (r, 0)) if two_acc else (lambda r: (0, 0))

    def body(a0_ref, a1_ref, xl_ref, xr_ref, epl_ref, att_ref, bias_ref,
             s_ref, st_ref, g1_ref, g2_ref, o_ref):
        xl = xl_ref[:, :HC]
        xr = xr_ref[:, :HC]
        m = xl + xr + epl_ref[...]
        m = jnp.where(m > 0, m, 0.2 * m)
        alpha = jnp.dot(m * att_ref[...], s_ref[...],
                        preferred_element_type=jnp.float32)
        ex = jnp.exp(alpha)
        acc = a0_ref[...] + a1_ref[...]
        exf = jnp.dot(ex, st_ref[...], preferred_element_type=jnp.float32)
        num = jnp.dot(acc, g1_ref[...],
                      preferred_element_type=jnp.float32) + exf * xl
        den = jnp.dot(acc, g2_ref[...],
                      preferred_element_type=jnp.float32) + ex
        denf = jnp.dot(den, st_ref[...], preferred_element_type=jnp.float32)
        y = num / (denf + _EPS) + bias_ref[...]
        o_ref[...] = jnp.where(y > 0, y, 0.01 * y)

    return pl.pallas_call(
        body,
        grid=(_grid_rows(n),),
        in_specs=[
            pl.BlockSpec((_BRF, PACC), lambda r: (r, 0)),
            pl.BlockSpec((_BRF, PACC), a1_map),
            pl.BlockSpec((_BRF, PXL), lambda r: (r, 0)),
            pl.BlockSpec((_BRF, PXR), lambda r: (r, 0)),
            pl.BlockSpec((1, HC), lambda r: (0, 0)),
            pl.BlockSpec((1, HC), lambda r: (0, 0)),
            pl.BlockSpec((1, HC), lambda r: (0, 0)),
            pl.BlockSpec((HC, H), lambda r: (0, 0)),
            pl.BlockSpec((H, HC), lambda r: (0, 0)),
            pl.BlockSpec((PACC, HC), lambda r: (0, 0)),
            pl.BlockSpec((PACC, H), lambda r: (0, 0)),
        ],
        out_specs=pl.BlockSpec((_BRF, HC), lambda r: (r, 0)),
        out_shape=jax.ShapeDtypeStruct((n, HC), jnp.float32),
    )(A0, A1, XL, XR, epl, att, bias, S, ST, G1, G2)


def _iface_pass(U, V, srcp, dstp, attrp, wflat, *, CHUNK, NCH, E):
    PU = U.shape[1]
    PWC = wflat.shape[0]
    EW = CHUNK * NCH

    @functools.partial(
        pl.kernel, mesh=_sc_mesh(),
        out_type=jax.ShapeDtypeStruct((_NW, _NL), jnp.float32),
        scratch_types=[
            [pltpu.VMEM((CHUNK,), jnp.int32)] * 2,
            [pltpu.VMEM((CHUNK,), jnp.int32)] * 2,
            [pltpu.VMEM((CHUNK,), jnp.float32)] * 2,
            [pltpu.VMEM((CHUNK, PU), jnp.float32)] * 2,
            [pltpu.VMEM((CHUNK, PU), jnp.float32)] * 2,
            pltpu.VMEM((PWC, _NL), jnp.float32),
            pltpu.VMEM((_NL,), jnp.float32),
            [pltpu.SemaphoreType.DMA] * 2,
            [pltpu.SemaphoreType.DMA] * 2,
        ],
        compiler_params=_SC_PARAMS)
    def k(u_hbm, v_hbm, src_hbm, dst_hbm, attr_hbm, wc_hbm, out_hbm,
          src_v, dst_v, attr_v, ur_v, vr_v, wc_v, o_v, sem1, sem2):
        cid = lax.axis_index("c")
        sid = lax.axis_index("s")
        wid = sid * _NC + cid
        iota = lax.iota(jnp.int32, _NL)
        pltpu.sync_copy(wc_hbm, wc_v)
        ebase = wid * EW

        def w69(j):
            return wc_v[j, :]

        def w2(j, kk):
            return wc_v[32 + j * 10 + kk, :]

        def b2(kk):
            return wc_v[282 + kk, :]

        def w3(kk):
            return wc_v[292 + kk, :]

        b3 = wc_v[302, :]

        def fire(ch, b):
            eb = ebase + ch * CHUNK
            pltpu.sync_copy(src_hbm.at[pl.ds(eb, CHUNK)], src_v[b])
            pltpu.sync_copy(dst_hbm.at[pl.ds(eb, CHUNK)], dst_v[b])
            pltpu.sync_copy(attr_hbm.at[pl.ds(eb, CHUNK)], attr_v[b])
            pltpu.async_copy(u_hbm.at[src_v[b], :], ur_v[b], sem1[b])
            pltpu.async_copy(v_hbm.at[dst_v[b], :], vr_v[b], sem2[b])

        def consume(ch, b, acc):
            eb = ebase + ch * CHUNK
            pltpu.make_async_copy(u_hbm.at[src_v[b], :], ur_v[b],
                                  sem1[b]).wait()
            pltpu.make_async_copy(v_hbm.at[dst_v[b], :], vr_v[b],
                                  sem2[b]).wait()

            def gb(g, acc2):
                rows = iota + g * 16
                eav = plsc.load_gather(attr_v[b], [rows])
                ts = []
                for j in range(25):
                    uv = plsc.load_gather(ur_v[b], [rows, _splat(j)])
                    vv = plsc.load_gather(vr_v[b], [rows, _splat(j)])
                    t = uv + vv + eav * w69(j)
                    ts.append(jnp.where(t > 0, t, 0.01 * t))
                s = b3
                for kk in range(10):
                    z = b2(kk)
                    for j in range(25):
                        z = z + ts[j] * w2(j, kk)
                    z = jnp.where(z > 0, z, 0.01 * z)
                    s = s + z * w3(kk)
                eid = eb + g * 16 + iota
                return acc2 + jnp.where(eid < E, s, 0.0)
            return lax.fori_loop(0, CHUNK // 16, gb, acc)

        fire(0, 0)
        fire(1, 1)

        def chunk_pair(i, acc):
            a = 2 * i
            acc = consume(a, 0, acc)
            fire(a + 2, 0)
            acc = consume(a + 1, 1, acc)
            fire(a + 3, 1)
            return acc
        acc = lax.fori_loop(0, NCH // 2 - 1, chunk_pair,
                            jnp.zeros((_NL,), jnp.float32))
        acc = consume(NCH - 2, 0, acc)
        acc = consume(NCH - 1, 1, acc)
        o_v[...] = acc
        pltpu.sync_copy(o_v, out_hbm.at[wid])

    return k(U, V, srcp, dstp, attrp, wflat)


def _final_sum(parts):
    def body(p_ref, o_ref):
        o_ref[...] = jnp.reshape(jnp.sum(p_ref[...]), (1, 1))

    return pl.pallas_call(
        body,
        out_shape=jax.ShapeDtypeStruct((1, 1), jnp.float32),
    )(parts)


def _gat_layer(y, srcp, dstp, attrp, mean_attr, p, *, H, C, PX, PACC, CHUNK,
               NCH, NACC, BR, npass, n):
    HC = H * C
    Wl = jnp.pad(p["Wl"], ((0, 0), (0, PX - HC)))
    bl = jnp.pad(p["bl"], (0, PX - HC))[None, :]
    Wr = jnp.pad(p["Wr"], ((0, 0), (0, PX - HC)))
    br = jnp.pad(p["br"], (0, PX - HC))[None, :]
    xl, xr = _mm2(y, Wl, bl, Wr, br)
    xlp = jnp.pad(xl, ((0, 1), (0, 0)))
    xrp = jnp.pad(xr, ((0, 1), (0, 0)))

    we_row = p["We"][0]
    att_row = p["att"][0].reshape(HC)
    wvec = jnp.repeat(jnp.concatenate([we_row, att_row])[:, None], _NL, axis=1)

    rng = n // npass
    accs = [_gat_edge_pass(xlp, xrp, srcp, dstp, attrp, wvec, H=H, C=C,
                           PACC=PACC, CHUNK=CHUNK, NCH=NCH, NACC=NACC, BR=BR,
                           lo=q * rng, rng=rng)
            for q in range(npass)]
    if npass == 1:
        A0, A1 = accs[0][0], accs[0][1]
    else:
        A0 = jnp.concatenate([a[0, :rng] for a in accs], axis=0)
        A1 = jnp.concatenate([a[1, :rng] for a in accs], axis=0)

    S = jnp.asarray(np.kron(np.eye(H), np.ones((C, 1))), jnp.float32)
    ST = S.T
    G1 = np.zeros((PACC, HC), np.float32)
    G2 = np.zeros((PACC, H), np.float32)
    for h in range(H):
        for c in range(C):
            G1[h * (C + 1) + c, h * C + c] = 1.0
        G2[h * (C + 1) + C, h] = 1.0
    G1 = jnp.asarray(G1)
    G2 = jnp.asarray(G2)

    epl = (mean_attr * we_row)[None, :]
    att2 = att_row[None, :]
    bias = p["bias"][None, :]
    return _finalize(A0, A1, xlp, xrp, epl, att2, bias, S, ST, G1, G2,
                     H=H, C=C, n=n, two_acc=True)


def kernel(x, peptide_bond_edge_index, peptide_bond_edge_attr,
           same_protein_edge_index, same_protein_edge_attr,
           interface_edge_index, interface_edge_attr, params):
    n = x.shape[0]

    def pad_edges(idx, attr, epad):
        e = idx.shape[1]
        src = jnp.pad(idx[0], (0, epad - e), constant_values=n)
        dst = jnp.pad(idx[1], (0, epad - e), constant_values=n)
        at = jnp.pad(attr.reshape(-1), (0, epad - e))
        return src, dst, at

    pb_src, pb_dst, pb_at = pad_edges(
        peptide_bond_edge_index, peptide_bond_edge_attr, 212992)
    sp_src, sp_dst, sp_at = pad_edges(
        same_protein_edge_index, same_protein_edge_attr, 1605632)
    if_src, if_dst, if_at = pad_edges(
        interface_edge_index, interface_edge_attr[:, None], 425984)

    pb_mean = jnp.mean(peptide_bond_edge_attr)
    sp_mean = jnp.mean(same_protein_edge_attr)

    y = _gat_layer(x, pb_src, pb_dst, pb_at, pb_mean, params["pc1"],
                   H=2, C=5, PX=16, PACC=16, CHUNK=256, NCH=26,
                   NACC=102400, BR=64, npass=1, n=n)
    y = _gat_layer(y, pb_src, pb_dst, pb_at, pb_mean, params["pc2"],
                   H=3, C=5, PX=16, PACC=24, CHUNK=256, NCH=26,
                   NACC=53248, BR=64, npass=2, n=n)
    px = _gat_layer(y, pb_src, pb_dst, pb_at, pb_mean, params["pc3"],
                    H=3, C=10, PX=32, PACC=40, CHUNK=256, NCH=26,
                    NACC=28672, BR=64, npass=4, n=n)
    y = _gat_layer(px, sp_src, sp_dst, sp_at, sp_mean, params["prc1"],
                   H=2, C=2, PX=16, PACC=16, CHUNK=256, NCH=196,
                   NACC=102400, BR=64, npass=1, n=n)
    prx = _gat_layer(y, sp_src, sp_dst, sp_at, sp_mean, params["prc2"],
                     H=2, C=2, PX=16, PACC=16, CHUNK=256, NCH=196,
                     NACC=102400, BR=64, npass=1, n=n)

    e1W = params["e1W"]
    Wa1 = jnp.pad(e1W[0:30], ((0, 0), (0, 7)))
    Wb1 = jnp.pad(e1W[30:34], ((0, 0), (0, 7)))
    b1 = jnp.pad(params["e1b"], (0, 7))[None, :]
    Wa2 = jnp.pad(e1W[34:64], ((0, 0), (0, 7)))
    Wb2 = jnp.pad(e1W[64:68], ((0, 0), (0, 7)))
    bz = jnp.zeros((1, 32), jnp.float32)
    U, V = _mm2b(px, prx, Wa1, Wb1, b1, Wa2, Wb2, bz)
    Up = jnp.pad(U, ((0, 1), (0, 0)))
    Vp = jnp.pad(V, ((0, 1), (0, 0)))

    wflat = jnp.zeros((304,), jnp.float32)
    wflat = wflat.at[0:25].set(e1W[68])
    wflat = wflat.at[32:282].set(params["e2W"].reshape(-1))
    wflat = wflat.at[282:292].set(params["e2b"])
    wflat = wflat.at[292:302].set(params["e3W"][:, 0])
    wflat = wflat.at[302].set(params["e3b"][0])
    wflat = jnp.repeat(wflat[:, None], _NL, axis=1)

    parts = _iface_pass(Up, Vp, if_src, if_dst, if_at, wflat,
                        CHUNK=512, NCH=26, E=interface_edge_index.shape[1])
    return _final_sum(parts)


# speedup vs baseline: 104.4889x; 1.0484x over previous
import functools

import jax
import jax.numpy as jnp
import numpy as np
from jax import lax
from jax.experimental import pallas as pl
from jax.experimental.pallas import tpu as pltpu
from jax.experimental.pallas import tpu_sc as plsc

_NC, _NS, _NL = 2, 16, 16
_NW = _NC * _NS

_SC_PARAMS = pltpu.CompilerParams(
    use_tc_tiling_on_sc=False, needs_layout_passes=False)

_EPS = 1e-16


def _pad16(n):
    return ((n + 15) // 16) * 16


def _splat(v):
    return jnp.broadcast_to(jnp.int32(v), (_NL,))


def _sc_mesh():
    return plsc.VectorSubcoreMesh(
        core_axis_name="c", subcore_axis_name="s",
        num_cores=_NC, num_subcores=_NS)


def _gat_edge_pass(xlp, xrp, srcp, dstp, attrp, wvec, *, H, C, PACC, CHUNK,
                   NCH, NACC, BR, lo, rng):
    HC = H * C
    PXL = xlp.shape[1]
    PXR = xrp.shape[1]
    NWV = wvec.shape[0]
    EW = CHUNK * NCH
    NB = NACC // _NS // BR

    @functools.partial(
        pl.kernel, mesh=_sc_mesh(),
        out_type=jax.ShapeDtypeStruct((_NC, NACC, PACC), jnp.float32),
        scratch_types=[
            [pltpu.VMEM((CHUNK,), jnp.int32)] * 2,
            [pltpu.VMEM((CHUNK,), jnp.int32)] * 2,
            [pltpu.VMEM((CHUNK,), jnp.int32)] * 2,
            [pltpu.VMEM((CHUNK,), jnp.float32)] * 2,
            [pltpu.VMEM((CHUNK, PXL), jnp.float32)] * 2,
            [pltpu.VMEM((CHUNK, PXR), jnp.float32)] * 2,
            [pltpu.VMEM((CHUNK, PACC), jnp.float32)] * 2,
            pltpu.VMEM((BR, PACC), jnp.float32),
            pltpu.VMEM((NWV, _NL), jnp.float32),
            pltpu.VMEM_SHARED((NACC, PACC), jnp.float32),
            [pltpu.SemaphoreType.DMA] * 2,
            [pltpu.SemaphoreType.DMA] * 2,
            [pltpu.SemaphoreType.DMA] * 2,
        ],
        compiler_params=_SC_PARAMS)
    def k(xl_hbm, xr_hbm, src_hbm, dst_hbm, attr_hbm, wv_hbm, out_hbm,
          src_v, dst_v, targ_v, attr_v, xlr_v, xrr_v, sc_v, bb_v, wv_v,
          acc_sh, sem1, sem2, sem3):
        cid = lax.axis_index("c")
        sid = lax.axis_index("s")
        iota = lax.iota(jnp.int32, _NL)
        zero16 = jnp.zeros((_NL,), jnp.float32)
        pltpu.sync_copy(wv_hbm, wv_v)

        if PACC >= 16:
            def zrow(r, _):
                for j0 in range(0, PACC, 16):
                    j = min(j0, PACC - 16)
                    bb_v[r, pl.ds(j, 16)] = zero16
                return 0
            lax.fori_loop(0, BR, zrow, 0)

            def zrow2(r, _):
                for b in range(2):
                    for j0 in range(0, PACC, 16):
                        j = min(j0, PACC - 16)
                        sc_v[b][r, pl.ds(j, 16)] = zero16
                return 0
            lax.fori_loop(0, CHUNK, zrow2, 0)
        else:
            rw = 16 // PACC

            def zrow(r, _):
                plsc.store_scatter(bb_v, [r * rw + iota // PACC, iota % PACC],
                                   zero16)
                return 0
            lax.fori_loop(0, BR // rw, zrow, 0)

            def zrow2(r, _):
                for b in range(2):
                    plsc.store_scatter(
                        sc_v[b], [r * rw + iota // PACC, iota % PACC], zero16)
                return 0
            lax.fori_loop(0, CHUNK // rw, zrow2, 0)

        base0 = sid * (NACC // _NS)

        def zacc(i, _):
            pltpu.sync_copy(bb_v, acc_sh.at[pl.ds(base0 + i * BR, BR)])
            return 0
        lax.fori_loop(0, NB, zacc, 0)
        plsc.subcore_barrier()

        def we_s(j):
            return wv_v[j, :]

        def att_s(j):
            return wv_v[HC + j, :]

        ebase = (sid * _NC + cid) * EW

        def fire(ch, b):
            eb = ebase + ch * CHUNK
            pltpu.sync_copy(src_hbm.at[pl.ds(eb, CHUNK)], src_v[b])
            pltpu.sync_copy(dst_hbm.at[pl.ds(eb, CHUNK)], dst_v[b])
            pltpu.sync_copy(attr_hbm.at[pl.ds(eb, CHUNK)], attr_v[b])
            pltpu.async_copy(xl_hbm.at[src_v[b], :], xlr_v[b], sem1[b])
            pltpu.async_copy(xr_hbm.at[dst_v[b], :], xrr_v[b], sem2[b])

        def waitsc(b):
            pltpu.make_async_copy(sc_v[b], acc_sh.at[targ_v[b], :],
                                  sem3[b]).wait()

        def consume(b):
            pltpu.make_async_copy(xl_hbm.at[src_v[b], :], xlr_v[b],
                                  sem1[b]).wait()
            pltpu.make_async_copy(xr_hbm.at[dst_v[b], :], xrr_v[b],
                                  sem2[b]).wait()

            def tb(g, _):
                rows = iota + g * 16
                t = plsc.load_gather(dst_v[b], [rows])
                inr = (t >= lo) & (t < lo + rng)
                tl = jnp.where(inr, t - lo, jnp.int32(NACC - 1))
                plsc.store_scatter(targ_v[b], [rows], tl)
                return 0
            lax.fori_loop(0, CHUNK // 16, tb, 0)

            def gb(g, _):
                rows = iota + g * 16
                eav = plsc.load_gather(attr_v[b], [rows])
                for h in range(H):
                    alpha = None
                    xls = []
                    for c in range(C):
                        j = h * C + c
                        xlv = plsc.load_gather(xlr_v[b], [rows, _splat(j)])
                        xrv = plsc.load_gather(xrr_v[b], [rows, _splat(j)])
                        m = xlv + xrv + eav * we_s(j)
                        m = jnp.where(m > 0, m, 0.2 * m)
                        a = m * att_s(j)
                        alpha = a if alpha is None else alpha + a
                        xls.append(xlv)
                    ex = jnp.exp(alpha)
                    for c in range(C):
                        plsc.store_scatter(
                            sc_v[b], [rows, _splat(h * (C + 1) + c)],
                            xls[c] * ex)
                    plsc.store_scatter(
                        sc_v[b], [rows, _splat(h * (C + 1) + C)], ex)
                return 0
            lax.fori_loop(0, CHUNK // 16, gb, 0)

            pltpu.async_copy(sc_v[b], acc_sh.at[targ_v[b], :], sem3[b],
                             add=True)

        fire(0, 0)
        fire(1, 1)
        consume(0)
        fire(2, 0)
        consume(1)
        fire(3, 1)

        def chunk_pair(i, _):
            a = 2 * i
            waitsc(0)
            consume(0)
            fire(a + 2, 0)
            waitsc(1)
            consume(1)
            fire(a + 3, 1)
            return 0
        lax.fori_loop(1, NCH // 2 - 1, chunk_pair, 0)
        waitsc(0)
        consume(0)
        waitsc(1)
        consume(1)
        waitsc(0)
        waitsc(1)
        plsc.subcore_barrier()

        def dump(i, _):
            r0 = base0 + i * BR
            pltpu.sync_copy(acc_sh.at[pl.ds(r0, BR)], bb_v)
            pltpu.sync_copy(bb_v, out_hbm.at[cid, pl.ds(r0, BR)])
            return 0
        lax.fori_loop(0, NB, dump, 0)

    return k(xlp, xrp, srcp, dstp, attrp, wvec)


_BRF = 2048


def _grid_rows(n):
    return (n + _BRF - 1) // _BRF


def _mm2(y, W1, b1, W2, b2):
    n, din = y.shape
    p1 = W1.shape[1]
    p2 = W2.shape[1]

    def body(y_ref, w1_ref, b1_ref, w2_ref, b2_ref, o1_ref, o2_ref):
        yb = y_ref[...]
        o1_ref[...] = jnp.dot(yb, w1_ref[...],
                              preferred_element_type=jnp.float32) + b1_ref[...]
        o2_ref[...] = jnp.dot(yb, w2_ref[...],
                              preferred_element_type=jnp.float32) + b2_ref[...]

    return pl.pallas_call(
        body,
        grid=(_grid_rows(n),),
        in_specs=[
            pl.BlockSpec((_BRF, din), lambda r: (r, 0)),
            pl.BlockSpec((din, p1), lambda r: (0, 0)),
            pl.BlockSpec((1, p1), lambda r: (0, 0)),
            pl.BlockSpec((din, p2), lambda r: (0, 0)),
            pl.BlockSpec((1, p2), lambda r: (0, 0)),
        ],
        out_specs=[
            pl.BlockSpec((_BRF, p1), lambda r: (r, 0)),
            pl.BlockSpec((_BRF, p2), lambda r: (r, 0)),
        ],
        out_shape=[
            jax.ShapeDtypeStruct((n, p1), jnp.float32),
            jax.ShapeDtypeStruct((n, p2), jnp.float32),
        ],
    )(y, W1, b1, W2, b2)


def _mm2b(a, b, Wa1, Wb1, b1, Wa2, Wb2, b2):
    n, da = a.shape
    db = b.shape[1]
    p1 = Wa1.shape[1]
    p2 = Wa2.shape[1]

    def body(a_ref, b_ref, wa1_ref, wb1_ref, b1_ref, wa2_ref, wb2_ref, b2_ref,
             o1_ref, o2_ref):
        ab = a_ref[...]
        bb = b_ref[...]
        o1_ref[...] = (jnp.dot(ab, wa1_ref[...], preferred_element_type=jnp.float32)
                       + jnp.dot(bb, wb1_ref[...], preferred_element_type=jnp.float32)
                       + b1_ref[...])
        o2_ref[...] = (jnp.dot(ab, wa2_ref[...], preferred_element_type=jnp.float32)
                       + jnp.dot(bb, wb2_ref[...], preferred_element_type=jnp.float32)
                       + b2_ref[...])

    return pl.pallas_call(
        body,
        grid=(_grid_rows(n),),
        in_specs=[
            pl.BlockSpec((_BRF, da), lambda r: (r, 0)),
            pl.BlockSpec((_BRF, db), lambda r: (r, 0)),
            pl.BlockSpec((da, p1), lambda r: (0, 0)),
            pl.BlockSpec((db, p1), lambda r: (0, 0)),
            pl.BlockSpec((1, p1), lambda r: (0, 0)),
            pl.BlockSpec((da, p2), lambda r: (0, 0)),
            pl.BlockSpec((db, p2), lambda r: (0, 0)),
            pl.BlockSpec((1, p2), lambda r: (0, 0)),
        ],
        out_specs=[
            pl.BlockSpec((_BRF, p1), lambda r: (r, 0)),
            pl.BlockSpec((_BRF, p2), lambda r: (r, 0)),
        ],
        out_shape=[
            jax.ShapeDtypeStruct((n, p1), jnp.float32),
            jax.ShapeDtypeStruct((n, p2), jnp.float32),
        ],
    )(a, b, Wa1, Wb1, b1, Wa2, Wb2, b2)


def _finalize(A0, A1, XL, XR, epl, att, bias, S, ST, G1, G2, *, H, C, n,
              two_acc):
    HC = H * C
    PACC = A0.shape[1]
    PXL = XL.shape[1]
    PXR = XR.shape[1]
    a1_map = (lambda r: (r, 0)) if two_acc else (lambda r: (0, 0))

    def body(a0_ref, a1_ref, xl_ref, xr_ref, epl_ref, att_ref, bias_ref,
             s_ref, st_ref, g1_ref, g2_ref, o_ref):
        xl = xl_ref[:, :HC]
        xr = xr_ref[:, :HC]
        m = xl + xr + epl_ref[...]
        m = jnp.where(m > 0, m, 0.2 * m)
        alpha = jnp.dot(m * att_ref[...], s_ref[...],
                        preferred_element_type=jnp.float32)
        ex = jnp.exp(alpha)
        acc = a0_ref[...] + a1_ref[...]
        exf = jnp.dot(ex, st_ref[...], preferred_element_type=jnp.float32)
        num = jnp.dot(acc, g1_ref[...],
                      preferred_element_type=jnp.float32) + exf * xl
        den = jnp.dot(acc, g2_ref[...],
                      preferred_element_type=jnp.float32) + ex
        denf = jnp.dot(den, st_ref[...], preferred_element_type=jnp.float32)
        y = num / (denf + _EPS) + bias_ref[...]
        o_ref[...] = jnp.where(y > 0, y, 0.01 * y)

    return pl.pallas_call(
        body,
        grid=(_grid_rows(n),),
        in_specs=[
            pl.BlockSpec((_BRF, PACC), lambda r: (r, 0)),
            pl.BlockSpec((_BRF, PACC), a1_map),
            pl.BlockSpec((_BRF, PXL), lambda r: (r, 0)),
            pl.BlockSpec((_BRF, PXR), lambda r: (r, 0)),
            pl.BlockSpec((1, HC), lambda r: (0, 0)),
            pl.BlockSpec((1, HC), lambda r: (0, 0)),
            pl.BlockSpec((1, HC), lambda r: (0, 0)),
            pl.BlockSpec((HC, H), lambda r: (0, 0)),
            pl.BlockSpec((H, HC), lambda r: (0, 0)),
            pl.BlockSpec((PACC, HC), lambda r: (0, 0)),
            pl.BlockSpec((PACC, H), lambda r: (0, 0)),
        ],
        out_specs=pl.BlockSpec((_BRF, HC), lambda r: (r, 0)),
        out_shape=jax.ShapeDtypeStruct((n, HC), jnp.float32),
    )(A0, A1, XL, XR, epl, att, bias, S, ST, G1, G2)


def _iface_pass(U, V, srcp, dstp, attrp, wflat, *, CHUNK, NCH, E):
    PU = U.shape[1]
    PWC = wflat.shape[0]
    EW = CHUNK * NCH

    @functools.partial(
        pl.kernel, mesh=_sc_mesh(),
        out_type=jax.ShapeDtypeStruct((_NW, _NL), jnp.float32),
        scratch_types=[
            [pltpu.VMEM((CHUNK,), jnp.int32)] * 2,
            [pltpu.VMEM((CHUNK,), jnp.int32)] * 2,
            [pltpu.VMEM((CHUNK,), jnp.float32)] * 2,
            [pltpu.VMEM((CHUNK, PU), jnp.float32)] * 2,
            [pltpu.VMEM((CHUNK, PU), jnp.float32)] * 2,
            pltpu.VMEM((PWC, _NL), jnp.float32),
            pltpu.VMEM((_NL,), jnp.float32),
            [pltpu.SemaphoreType.DMA] * 2,
            [pltpu.SemaphoreType.DMA] * 2,
        ],
        compiler_params=_SC_PARAMS)
    def k(u_hbm, v_hbm, src_hbm, dst_hbm, attr_hbm, wc_hbm, out_hbm,
          src_v, dst_v, attr_v, ur_v, vr_v, wc_v, o_v, sem1, sem2):
        cid = lax.axis_index("c")
        sid = lax.axis_index("s")
        wid = sid * _NC + cid
        iota = lax.iota(jnp.int32, _NL)
        pltpu.sync_copy(wc_hbm, wc_v)
        ebase = wid * EW

        def w69(j):
            return wc_v[j, :]

        def w2(j, kk):
            return wc_v[32 + j * 10 + kk, :]

        def b2(kk):
            return wc_v[282 + kk, :]

        def w3(kk):
            return wc_v[292 + kk, :]

        b3 = wc_v[302, :]

        def fire(ch, b):
            eb = ebase + ch * CHUNK
            pltpu.sync_copy(src_hbm.at[pl.ds(eb, CHUNK)], src_v[b])
            pltpu.sync_copy(dst_hbm.at[pl.ds(eb, CHUNK)], dst_v[b])
            pltpu.sync_copy(attr_hbm.at[pl.ds(eb, CHUNK)], attr_v[b])
            pltpu.async_copy(u_hbm.at[src_v[b], :], ur_v[b], sem1[b])
            pltpu.async_copy(v_hbm.at[dst_v[b], :], vr_v[b], sem2[b])

        def consume(ch, b, acc):
            eb = ebase + ch * CHUNK
            pltpu.make_async_copy(u_hbm.at[src_v[b], :], ur_v[b],
                                  sem1[b]).wait()
            pltpu.make_async_copy(v_hbm.at[dst_v[b], :], vr_v[b],
                                  sem2[b]).wait()

            def gb(g, acc2):
                rows = iota + g * 16
                eav = plsc.load_gather(attr_v[b], [rows])
                ts = []
                for j in range(25):
                    uv = plsc.load_gather(ur_v[b], [rows, _splat(j)])
                    vv = plsc.load_gather(vr_v[b], [rows, _splat(j)])
                    t = uv + vv + eav * w69(j)
                    ts.append(jnp.where(t > 0, t, 0.01 * t))
                s = b3
                for kk in range(10):
                    z = b2(kk)
                    for j in range(25):
                        z = z + ts[j] * w2(j, kk)
                    z = jnp.where(z > 0, z, 0.01 * z)
                    s = s + z * w3(kk)
                eid = eb + g * 16 + iota
                return acc2 + jnp.where(eid < E, s, 0.0)
            return lax.fori_loop(0, CHUNK // 16, gb, acc)

        fire(0, 0)
        fire(1, 1)

        def chunk_pair(i, acc):
            a = 2 * i
            acc = consume(a, 0, acc)
            fire(a + 2, 0)
            acc = consume(a + 1, 1, acc)
            fire(a + 3, 1)
            return acc
        acc = lax.fori_loop(0, NCH // 2 - 1, chunk_pair,
                            jnp.zeros((_NL,), jnp.float32))
        acc = consume(NCH - 2, 0, acc)
        acc = consume(NCH - 1, 1, acc)
        o_v[...] = acc
        pltpu.sync_copy(o_v, out_hbm.at[wid])

    return k(U, V, srcp, dstp, attrp, wflat)


def _final_sum(parts):
    def body(p_ref, o_ref):
        o_ref[...] = jnp.reshape(jnp.sum(p_ref[...]), (1, 1))

    return pl.pallas_call(
        body,
        out_shape=jax.ShapeDtypeStruct((1, 1), jnp.float32),
    )(parts)


def _gat_layer(y, srcp, dstp, attrp, mean_attr, p, *, H, C, PX, PACC, CHUNK,
               NCH, NACC, BR, npass, n):
    HC = H * C
    Wl = jnp.pad(p["Wl"], ((0, 0), (0, PX - HC)))
    bl = jnp.pad(p["bl"], (0, PX - HC))[None, :]
    Wr = jnp.pad(p["Wr"], ((0, 0), (0, PX - HC)))
    br = jnp.pad(p["br"], (0, PX - HC))[None, :]
    xl, xr = _mm2(y, Wl, bl, Wr, br)
    xlp = jnp.pad(xl, ((0, 1), (0, 0)))
    xrp = jnp.pad(xr, ((0, 1), (0, 0)))

    we_row = p["We"][0]
    att_row = p["att"][0].reshape(HC)
    wvec = jnp.repeat(jnp.concatenate([we_row, att_row])[:, None], _NL, axis=1)

    rng = n // npass
    accs = [_gat_edge_pass(xlp, xrp, srcp, dstp, attrp, wvec, H=H, C=C,
                           PACC=PACC, CHUNK=CHUNK, NCH=NCH, NACC=NACC, BR=BR,
                           lo=q * rng, rng=rng)
            for q in range(npass)]
    if npass == 1:
        A0, A1 = accs[0][0], accs[0][1]
    else:
        A0 = jnp.concatenate([a[0, :rng] for a in accs], axis=0)
        A1 = jnp.concatenate([a[1, :rng] for a in accs], axis=0)

    S = jnp.asarray(np.kron(np.eye(H), np.ones((C, 1))), jnp.float32)
    ST = S.T
    G1 = np.zeros((PACC, HC), np.float32)
    G2 = np.zeros((PACC, H), np.float32)
    for h in range(H):
        for c in range(C):
            G1[h * (C + 1) + c, h * C + c] = 1.0
        G2[h * (C + 1) + C, h] = 1.0
    G1 = jnp.asarray(G1)
    G2 = jnp.asarray(G2)

    epl = (mean_attr * we_row)[None, :]
    att2 = att_row[None, :]
    bias = p["bias"][None, :]
    return _finalize(A0, A1, xlp, xrp, epl, att2, bias, S, ST, G1, G2,
                     H=H, C=C, n=n, two_acc=True)


def kernel(x, peptide_bond_edge_index, peptide_bond_edge_attr,
           same_protein_edge_index, same_protein_edge_attr,
           interface_edge_index, interface_edge_attr, params):
    n = x.shape[0]

    def pad_edges(idx, attr, epad):
        e = idx.shape[1]
        src = jnp.pad(idx[0], (0, epad - e), constant_values=n)
        dst = jnp.pad(idx[1], (0, epad - e), constant_values=n)
        at = jnp.pad(attr.reshape(-1), (0, epad - e))
        return src, dst, at

    pb_src, pb_dst, pb_at = pad_edges(
        peptide_bond_edge_index, peptide_bond_edge_attr, 212992)
    sp_src, sp_dst, sp_at = pad_edges(
        same_protein_edge_index, same_protein_edge_attr, 1605632)
    if_src, if_dst, if_at = pad_edges(
        interface_edge_index, interface_edge_attr[:, None], 425984)

    pb_mean = jnp.mean(peptide_bond_edge_attr)
    sp_mean = jnp.mean(same_protein_edge_attr)

    y = _gat_layer(x, pb_src, pb_dst, pb_at, pb_mean, params["pc1"],
                   H=2, C=5, PX=16, PACC=16, CHUNK=256, NCH=26,
                   NACC=102400, BR=64, npass=1, n=n)
    y = _gat_layer(y, pb_src, pb_dst, pb_at, pb_mean, params["pc2"],
                   H=3, C=5, PX=16, PACC=24, CHUNK=256, NCH=26,
                   NACC=53248, BR=64, npass=2, n=n)
    px = _gat_layer(y, pb_src, pb_dst, pb_at, pb_mean, params["pc3"],
                    H=3, C=10, PX=32, PACC=40, CHUNK=256, NCH=26,
                    NACC=28672, BR=64, npass=4, n=n)
    y = _gat_layer(px, sp_src, sp_dst, sp_at, sp_mean, params["prc1"],
                   H=2, C=2, PX=16, PACC=8, CHUNK=512, NCH=98,
                   NACC=102400, BR=64, npass=1, n=n)
    prx = _gat_layer(y, sp_src, sp_dst, sp_at, sp_mean, params["prc2"],
                     H=2, C=2, PX=16, PACC=8, CHUNK=512, NCH=98,
                     NACC=102400, BR=64, npass=1, n=n)

    e1W = params["e1W"]
    Wa1 = jnp.pad(e1W[0:30], ((0, 0), (0, 7)))
    Wb1 = jnp.pad(e1W[30:34], ((0, 0), (0, 7)))
    b1 = jnp.pad(params["e1b"], (0, 7))[None, :]
    Wa2 = jnp.pad(e1W[34:64], ((0, 0), (0, 7)))
    Wb2 = jnp.pad(e1W[64:68], ((0, 0), (0, 7)))
    bz = jnp.zeros((1, 32), jnp.float32)
    U, V = _mm2b(px, prx, Wa1, Wb1, b1, Wa2, Wb2, bz)
    Up = jnp.pad(U, ((0, 1), (0, 0)))
    Vp = jnp.pad(V, ((0, 1), (0, 0)))

    wflat = jnp.zeros((304,), jnp.float32)
    wflat = wflat.at[0:25].set(e1W[68])
    wflat = wflat.at[32:282].set(params["e2W"].reshape(-1))
    wflat = wflat.at[282:292].set(params["e2b"])
    wflat = wflat.at[292:302].set(params["e3W"][:, 0])
    wflat = wflat.at[302].set(params["e3b"][0])
    wflat = jnp.repeat(wflat[:, None], _NL, axis=1)

    parts = _iface_pass(Up, Vp, if_src, if_dst, if_at, wflat,
                        CHUNK=512, NCH=26, E=interface_edge_index.shape[1])
    return _final_sum(parts)
